# baseline (speedup 1.0000x reference)
import functools

import jax
import jax.numpy as jnp
from jax import lax
from jax.experimental import pallas as pl
from jax.experimental.pallas import tpu as pltpu
from jax.experimental.pallas import tpu_sc as plsc

NUM_CORES = 2
NUM_SUBCORES = 16
NW = NUM_CORES * NUM_SUBCORES

BLK = 128
L = 16


def _emb_body(seq, d, ids_hbm, table_hbm, out_hbm,
              idx_v, idx2_v, rows0, rows1, t0, t1,
              gsem0, gsem1, osem0, osem1):
  wid = lax.axis_index("s") * NUM_CORES + lax.axis_index("c")

  pltpu.sync_copy(ids_hbm.at[:, pl.ds(wid * BLK, BLK)], idx_v)

  lanes = lax.iota(jnp.int32, L)

  def prep_idx(s, slot):
    for j in range(BLK // L):
      v = idx_v[s, pl.ds(j * L, L)]
      idx2_v[slot, 0, pl.ds(j * L, L)] = lax.shift_right_logical(v, 1)
      idx2_v[slot, 1, pl.ds(j * L, L)] = lax.shift_left(
          lax.bitwise_and(v, 1), 6)

  def fire_gather(slot, rows_v, gsem):
    return pltpu.async_copy(
        table_hbm.at[idx2_v.at[slot, 0]], rows_v, gsem)

  def transpose_into(rows_v, t_v, slot):
    def dbody(dd, carry):
      for j in range(BLK // L):
        row_i = lanes + (j * L)
        col_i = idx2_v[slot, 1, pl.ds(j * L, L)] + dd
        vals = plsc.load_gather(rows_v, [row_i, col_i])
        t_v[dd, pl.ds(j * L, L)] = vals
      return carry
    lax.fori_loop(0, d, dbody, 0)

  def fire_out(s, t_v, osem):
    return pltpu.async_copy(
        t_v, out_hbm.at[s, :, pl.ds(wid * BLK, BLK)], osem)

  def wait_out(s, t_v, osem):
    pltpu.make_async_copy(
        t_v, out_hbm.at[s, :, pl.ds(wid * BLK, BLK)], osem).wait()

  def drain_gather(slot, rows_v, gsem):
    pltpu.make_async_copy(
        table_hbm.at[idx2_v.at[slot, 0]], rows_v, gsem).wait()

  prep_idx(0, 0)
  fire_gather(0, rows0, gsem0)
  prep_idx(1, 1)
  fire_gather(1, rows1, gsem1)

  drain_gather(0, rows0, gsem0)
  transpose_into(rows0, t0, 0)
  fire_out(0, t0, osem0)
  prep_idx(2, 0)
  fire_gather(0, rows0, gsem0)
  drain_gather(1, rows1, gsem1)
  transpose_into(rows1, t1, 1)
  fire_out(1, t1, osem1)
  prep_idx(3, 1)
  fire_gather(1, rows1, gsem1)

  def pair_body(i, carry):
    s = 2 * i
    drain_gather(0, rows0, gsem0)
    wait_out(s - 2, t0, osem0)
    transpose_into(rows0, t0, 0)
    fire_out(s, t0, osem0)
    prep_idx(lax.min(s + 2, seq - 1), 0)
    fire_gather(0, rows0, gsem0)
    drain_gather(1, rows1, gsem1)
    wait_out(s - 1, t1, osem1)
    transpose_into(rows1, t1, 1)
    fire_out(s + 1, t1, osem1)
    prep_idx(lax.min(s + 3, seq - 1), 1)
    fire_gather(1, rows1, gsem1)
    return carry

  lax.fori_loop(1, seq // 2, pair_body, 0)

  drain_gather(0, rows0, gsem0)
  drain_gather(1, rows1, gsem1)
  wait_out(seq - 2, t0, osem0)
  wait_out(seq - 1, t1, osem1)


@jax.jit
def kernel(input_ids, table):
  batch, seq = input_ids.shape
  n_rows, d = table.shape
  assert batch % (NW * BLK) == 0 or batch == NW * BLK

  ids_t = jnp.transpose(input_ids)
  tbl2 = jnp.reshape(table, (n_rows // 2, 2 * d))

  mesh = plsc.VectorSubcoreMesh(core_axis_name="c", subcore_axis_name="s")
  run = pl.kernel(
      functools.partial(_emb_body, seq, d),
      out_type=jax.ShapeDtypeStruct((seq, d, batch), jnp.float32),
      mesh=mesh,
      compiler_params=pltpu.CompilerParams(use_tc_tiling_on_sc=True,
                                           needs_layout_passes=False),
      scratch_types=[
          pltpu.VMEM((seq, BLK), jnp.int32),
          pltpu.VMEM((2, 2, BLK), jnp.int32),
          pltpu.VMEM((BLK, 2 * d), jnp.float32),
          pltpu.VMEM((BLK, 2 * d), jnp.float32),
          pltpu.VMEM((d, BLK), jnp.float32),
          pltpu.VMEM((d, BLK), jnp.float32),
          pltpu.SemaphoreType.DMA,
          pltpu.SemaphoreType.DMA,
          pltpu.SemaphoreType.DMA,
          pltpu.SemaphoreType.DMA,
      ],
  )
  out_t = run(ids_t, tbl2)
  return jnp.transpose(out_t, (2, 0, 1))


# speedup vs baseline: 2.4358x; 2.4358x over previous
import functools

import jax
import jax.numpy as jnp
from jax import lax
from jax.experimental import pallas as pl
from jax.experimental.pallas import tpu as pltpu
from jax.experimental.pallas import tpu_sc as plsc

NUM_CORES = 2
NUM_SUBCORES = 16
NW = NUM_CORES * NUM_SUBCORES

BLK = 128
L = 16


def _emb_body(seq, d, ids_hbm, table_hbm, out_hbm,
              idx_v, idx2_v, rows0, rows1, t0, t1,
              gsem0, gsem1, osem0, osem1):
  wid = lax.axis_index("s") * NUM_CORES + lax.axis_index("c")

  pltpu.sync_copy(ids_hbm.at[:, pl.ds(wid * BLK, BLK)], idx_v)

  lanes = lax.iota(jnp.int32, L)
  rows_j = [lanes + (j * L) for j in range(BLK // L)]

  def prep_idx(s, slot):
    for j in range(BLK // L):
      v = idx_v[s, pl.ds(j * L, L)]
      idx2_v[slot, 0, pl.ds(j * L, L)] = lax.shift_right_logical(v, 1)
      idx2_v[slot, 1, pl.ds(j * L, L)] = lax.shift_left(
          lax.bitwise_and(v, 1), 6)

  def fire_gather(slot, rows_v, gsem):
    return pltpu.async_copy(
        table_hbm.at[idx2_v.at[slot, 0]], rows_v, gsem)

  def transpose_into(rows_v, t_v, slot):
    and64 = [idx2_v[slot, 1, pl.ds(j * L, L)] for j in range(BLK // L)]

    def dbody(dd, carry):
      dmod = lax.bitwise_and(lanes + dd, d - 1)
      for j in range(BLK // L):
        vals = plsc.load_gather(rows_v, [rows_j[j], and64[j] + dmod])
        plsc.store_scatter(t_v, [dmod, rows_j[j]], vals)
      return carry
    lax.fori_loop(0, d, dbody, 0)

  def fire_out(s, t_v, osem):
    return pltpu.async_copy(
        t_v, out_hbm.at[s, :, pl.ds(wid * BLK, BLK)], osem)

  def wait_out(s, t_v, osem):
    pltpu.make_async_copy(
        t_v, out_hbm.at[s, :, pl.ds(wid * BLK, BLK)], osem).wait()

  def drain_gather(slot, rows_v, gsem):
    pltpu.make_async_copy(
        table_hbm.at[idx2_v.at[slot, 0]], rows_v, gsem).wait()

  prep_idx(0, 0)
  fire_gather(0, rows0, gsem0)
  prep_idx(1, 1)
  fire_gather(1, rows1, gsem1)

  drain_gather(0, rows0, gsem0)
  transpose_into(rows0, t0, 0)
  fire_out(0, t0, osem0)
  prep_idx(2, 0)
  fire_gather(0, rows0, gsem0)
  drain_gather(1, rows1, gsem1)
  transpose_into(rows1, t1, 1)
  fire_out(1, t1, osem1)
  prep_idx(3, 1)
  fire_gather(1, rows1, gsem1)

  def pair_body(i, carry):
    s = 2 * i
    drain_gather(0, rows0, gsem0)
    wait_out(s - 2, t0, osem0)
    transpose_into(rows0, t0, 0)
    fire_out(s, t0, osem0)
    prep_idx(lax.min(s + 2, seq - 1), 0)
    fire_gather(0, rows0, gsem0)
    drain_gather(1, rows1, gsem1)
    wait_out(s - 1, t1, osem1)
    transpose_into(rows1, t1, 1)
    fire_out(s + 1, t1, osem1)
    prep_idx(lax.min(s + 3, seq - 1), 1)
    fire_gather(1, rows1, gsem1)
    return carry

  lax.fori_loop(1, seq // 2, pair_body, 0)

  drain_gather(0, rows0, gsem0)
  drain_gather(1, rows1, gsem1)
  wait_out(seq - 2, t0, osem0)
  wait_out(seq - 1, t1, osem1)


@jax.jit
def kernel(input_ids, table):
  batch, seq = input_ids.shape
  n_rows, d = table.shape
  assert batch % (NW * BLK) == 0 or batch == NW * BLK

  ids_t = jnp.transpose(input_ids)
  tbl2 = jnp.reshape(table, (n_rows // 2, 2 * d))

  mesh = plsc.VectorSubcoreMesh(core_axis_name="c", subcore_axis_name="s")
  run = pl.kernel(
      functools.partial(_emb_body, seq, d),
      out_type=jax.ShapeDtypeStruct((seq, d, batch), jnp.float32),
      mesh=mesh,
      compiler_params=pltpu.CompilerParams(use_tc_tiling_on_sc=True,
                                           needs_layout_passes=False),
      scratch_types=[
          pltpu.VMEM((seq, BLK), jnp.int32),
          pltpu.VMEM((2, 2, BLK), jnp.int32),
          pltpu.VMEM((BLK, 2 * d), jnp.float32),
          pltpu.VMEM((BLK, 2 * d), jnp.float32),
          pltpu.VMEM((d, BLK), jnp.float32),
          pltpu.VMEM((d, BLK), jnp.float32),
          pltpu.SemaphoreType.DMA,
          pltpu.SemaphoreType.DMA,
          pltpu.SemaphoreType.DMA,
          pltpu.SemaphoreType.DMA,
      ],
  )
  out_t = run(ids_t, tbl2)
  return jnp.transpose(out_t, (2, 0, 1))
